# baseline (speedup 1.0000x reference)
import numpy as np

import jax
import jax.numpy as jnp
from jax import lax
from jax.experimental import pallas as pl
from jax.experimental.pallas import tpu as pltpu
from jax.experimental.pallas import tpu_sc as plsc

B = 4096
L = 200
D = 64
VOCAB = 1000000

NC = 2
NS = 16
NW = NC * NS
BPW = B // NW
LANES = 16

GW = 80
CHUNK_ITEMS = 2
CHUNK_ROWS = CHUNK_ITEMS * L
NG = CHUNK_ROWS // GW
NSG = BPW // LANES
NR = LANES // CHUNK_ITEMS

_PERM = np.concatenate([
    np.concatenate([np.arange(0, 32, 2), np.arange(1, 32, 2)]) + 32 * blk
    for blk in range(2)
])
_INV = np.argsort(_PERM)


def _round_copies(table_hbm, idx_v, rows, sem, row0):
    return [pltpu.make_async_copy(
        table_hbm.at[idx_v.at[pl.ds(row0 + gi * GW, GW)]],
        rows.at[pl.ds(gi * GW, GW)],
        sem) for gi in range(NG)]


def _bow_body(x_hbm, len_hbm, table_hbm, out_hbm,
              idx_v, rows0, rows1, out_v, len_v, sem0, sem1):
    cid = lax.axis_index("c")
    sid = lax.axis_index("s")
    wid = sid * NC + cid
    base = wid * BPW
    bufs = (rows0, rows1)
    sems = (sem0, sem1)

    pltpu.sync_copy(x_hbm.at[pl.ds(base * L, BPW * L)], idx_v)
    pltpu.sync_copy(len_hbm.at[pl.ds(base, BPW)], len_v)

    for cp in _round_copies(table_hbm, idx_v, bufs[0], sems[0], 0):
        cp.start()

    def sg_body(g, _):
        rv = 1.0 / len_v[pl.ds(g * LANES, LANES)].astype(jnp.float32)
        for r in range(NR):
            buf = r % 2
            if r < NR - 1:
                nxt = (g * LANES + (r + 1) * CHUNK_ITEMS) * L
                for cp in _round_copies(table_hbm, idx_v,
                                        bufs[1 - buf], sems[1 - buf], nxt):
                    cp.start()
            else:
                @pl.when(g < NSG - 1)
                def _():
                    nxt = (g + 1) * LANES * L
                    for cp in _round_copies(table_hbm, idx_v,
                                            bufs[1 - buf], sems[1 - buf],
                                            nxt):
                        cp.start()

            for cp in _round_copies(table_hbm, idx_v, bufs[buf], sems[buf],
                                    0):
                cp.wait()

            rows_v = bufs[buf]
            for it in range(CHUNK_ITEMS):
                lane = r * CHUNK_ITEMS + it
                item = g * LANES + lane

                def red_body(l, acc, it=it, rows_v=rows_v):
                    row = it * L + l
                    h0 = plsc.unpack(rows_v[row, pl.ds(0, 32)],
                                     format=plsc.PackFormat.INTERLEAVED)
                    h1 = plsc.unpack(rows_v[row, pl.ds(32, 32)],
                                     format=plsc.PackFormat.INTERLEAVED)
                    parts = (h0[0], h0[1], h1[0], h1[1])
                    return tuple(a + p for a, p in zip(acc, parts))

                acc0 = tuple(jnp.zeros((LANES,), jnp.float32)
                             for _ in range(4))
                acc = lax.fori_loop(0, L, red_body, acc0, unroll=8)
                scale = jnp.full((LANES,), rv[lane], jnp.float32)
                for j in range(4):
                    out_v[item, pl.ds(j * LANES, LANES)] = acc[j] * scale
        return 0

    lax.fori_loop(0, NSG, sg_body, 0)
    pltpu.sync_copy(out_v, out_hbm.at[pl.ds(base, BPW)])


@jax.jit
def kernel(x, length, emb_weight):
    x_flat = jnp.maximum(x.reshape(B * L).astype(jnp.int32), 0) * 2
    length = length.astype(jnp.int32)
    emb_bf = emb_weight.astype(jnp.bfloat16)
    emb2 = jnp.pad(emb_bf, ((0, 0), (0, 64))).reshape(2 * VOCAB, D)

    mesh = plsc.VectorSubcoreMesh(core_axis_name="c", subcore_axis_name="s")
    run = pl.kernel(
        _bow_body,
        out_type=jax.ShapeDtypeStruct((B, D), jnp.float32),
        mesh=mesh,
        scratch_types=[
            pltpu.VMEM((BPW * L,), jnp.int32),
            pltpu.VMEM((CHUNK_ROWS, D), jnp.bfloat16),
            pltpu.VMEM((CHUNK_ROWS, D), jnp.bfloat16),
            pltpu.VMEM((BPW, D), jnp.float32),
            pltpu.VMEM((BPW,), jnp.int32),
            pltpu.SemaphoreType.DMA,
            pltpu.SemaphoreType.DMA,
        ],
        compiler_params=pltpu.CompilerParams(use_tc_tiling_on_sc=False,
                                             needs_layout_passes=False),
    )
    out_perm = run(x_flat, length, emb2)
    return jnp.take(out_perm, jnp.asarray(_INV, dtype=jnp.int32), axis=1)


# speedup vs baseline: 2.0985x; 2.0985x over previous
import jax
import jax.numpy as jnp
from jax import lax
from jax.experimental import pallas as pl
from jax.experimental.pallas import tpu as pltpu
from jax.experimental.pallas import tpu_sc as plsc

B = 4096
L = 200
D = 64
VOCAB = 1000000

NC = 2
NS = 16
NW = NC * NS
BPW = B // NW
LANES = 16
DV = D // LANES

GW = 80
CHUNK_ITEMS = 2
CHUNK_ROWS = CHUNK_ITEMS * L
NG = CHUNK_ROWS // GW
NCHUNK = BPW // CHUNK_ITEMS


NSG = BPW // LANES
NR = LANES // CHUNK_ITEMS


def _round_copies(table_hbm, idx_v, rows, sem, row0):
    return [pltpu.make_async_copy(
        table_hbm.at[idx_v.at[pl.ds(row0 + gi * GW, GW)]],
        rows.at[pl.ds(gi * GW, GW)],
        sem) for gi in range(NG)]


def _bow_body(x_hbm, len_hbm, table_hbm, out_hbm,
              idx_v, rows0, rows1, out_v, len_v, sem0, sem1):
    cid = lax.axis_index("c")
    sid = lax.axis_index("s")
    wid = sid * NC + cid
    base = wid * BPW
    bufs = (rows0, rows1)
    sems = (sem0, sem1)

    pltpu.sync_copy(x_hbm.at[pl.ds(base * L, BPW * L)], idx_v)
    pltpu.sync_copy(len_hbm.at[pl.ds(base, BPW)], len_v)

    for cp in _round_copies(table_hbm, idx_v, bufs[0], sems[0], 0):
        cp.start()

    def sg_body(g, _):
        rv = 1.0 / len_v[pl.ds(g * LANES, LANES)].astype(jnp.float32)
        for r in range(NR):
            buf = r % 2
            if r < NR - 1:
                nxt = (g * LANES + (r + 1) * CHUNK_ITEMS) * L
                for cp in _round_copies(table_hbm, idx_v,
                                        bufs[1 - buf], sems[1 - buf], nxt):
                    cp.start()
            else:
                @pl.when(g < NSG - 1)
                def _():
                    nxt = (g + 1) * LANES * L
                    for cp in _round_copies(table_hbm, idx_v,
                                            bufs[1 - buf], sems[1 - buf],
                                            nxt):
                        cp.start()

            for cp in _round_copies(table_hbm, idx_v, bufs[buf], sems[buf],
                                    0):
                cp.wait()

            rows_v = bufs[buf]
            for it in range(CHUNK_ITEMS):
                lane = r * CHUNK_ITEMS + it
                item = g * LANES + lane

                def red_body(l, acc, it=it, rows_v=rows_v):
                    row = it * L + l
                    return tuple(
                        acc[j] + rows_v[row, pl.ds(j * LANES, LANES)]
                        for j in range(DV))

                acc0 = tuple(jnp.zeros((LANES,), jnp.float32)
                             for _ in range(DV))
                acc = lax.fori_loop(0, L, red_body, acc0, unroll=8)
                scale = jnp.full((LANES,), rv[lane], jnp.float32)
                for j in range(DV):
                    out_v[item, pl.ds(j * LANES, LANES)] = acc[j] * scale
        return 0

    lax.fori_loop(0, NSG, sg_body, 0)
    pltpu.sync_copy(out_v, out_hbm.at[pl.ds(base, BPW)])


@jax.jit
def kernel(x, length, emb_weight):
    x_flat = jnp.maximum(x.reshape(B * L).astype(jnp.int32), 0) * 2
    length = length.astype(jnp.int32)
    emb2 = jnp.pad(emb_weight, ((0, 0), (0, 64))).reshape(2 * VOCAB, D)

    mesh = plsc.VectorSubcoreMesh(core_axis_name="c", subcore_axis_name="s")
    run = pl.kernel(
        _bow_body,
        out_type=jax.ShapeDtypeStruct((B, D), jnp.float32),
        mesh=mesh,
        scratch_types=[
            pltpu.VMEM((BPW * L,), jnp.int32),
            pltpu.VMEM((CHUNK_ROWS, D), jnp.float32),
            pltpu.VMEM((CHUNK_ROWS, D), jnp.float32),
            pltpu.VMEM((BPW, D), jnp.float32),
            pltpu.VMEM((BPW,), jnp.int32),
            pltpu.SemaphoreType.DMA,
            pltpu.SemaphoreType.DMA,
        ],
        compiler_params=pltpu.CompilerParams(use_tc_tiling_on_sc=False,
                                             needs_layout_passes=False),
    )
    return run(x_flat, length, emb2)
